# baseline (speedup 1.0000x reference)
import functools

import jax
import jax.numpy as jnp
from jax import lax
from jax.experimental import pallas as pl
from jax.experimental.pallas import tpu as pltpu
from jax.experimental.pallas import tpu_sc as plsc

K = 1024
D = 64
DECAY = 0.99
EPSILON = 1e-05

NC = 2
NS = 16
NW = NC * NS
CH = 128
W = 128


def _tc_body(x_ref, cbt_ref, disc_ref, idx_ref, xe_ref, *, T):
    xb = x_ref[...]
    cbt = cbt_ref[...]

    x2 = jnp.sum(xb * xb, axis=1, keepdims=True)
    c2 = jnp.sum(cbt * cbt, axis=0, keepdims=True)
    dot = jax.lax.dot_general(xb, cbt, (((1,), (0,)), ((), ())),
                              preferred_element_type=jnp.float32)
    d = (x2 + (-2.0) * dot) + c2

    m = jnp.min(d, axis=1, keepdims=True)
    lane = jax.lax.broadcasted_iota(jnp.int32, (T, K), 1)
    cand = jnp.where(d == m, lane, K)
    idxc = jnp.min(cand, axis=1)[:, None]
    idx_ref[...] = idxc
    disc_ref[...] = (jax.lax.broadcasted_iota(jnp.int32, (T, K), 1)
                     == idxc).astype(jnp.float32)
    ones = jnp.ones((T, 1), jnp.float32)
    zeros = jnp.zeros((T, W - D - 1), jnp.float32)
    xe_ref[...] = jnp.concatenate([xb, ones, zeros], axis=1)


def _sc_body(xe_hbm, cb_hbm, idx_hbm, zw_hbm,
             quant_hbm, wsum_out,
             ir_v, xev, rows_v,
             wsum_sh, sem_l, sem_g, sem_s, *, nch, per_w):
    cid = lax.axis_index("c")
    sid = lax.axis_index("s")
    wid = sid * NC + cid

    @pl.when(sid == 0)
    def _init():
        pltpu.sync_copy(zw_hbm, wsum_sh)

    plsc.subcore_barrier()

    def fire_loads(j, b):
        base = wid * per_w + j * CH
        return [
            pltpu.async_copy(idx_hbm.at[pl.ds(base, CH)], ir_v[b], sem_l[b]),
            pltpu.async_copy(xe_hbm.at[pl.ds(base, CH)], xev[b], sem_l[b]),
        ]

    loads = fire_loads(0, 0)
    stores = []
    for j in range(nch):
        b = j & 1
        base = wid * per_w + j * CH
        for c in loads:
            c.wait()
        g1 = pltpu.async_copy(cb_hbm.at[ir_v[b]], rows_v[b], sem_g[b])
        for c in stores:
            c.wait()
        if j + 1 < nch:
            loads = fire_loads(j + 1, b ^ 1)
        g1.wait()
        stores = [
            pltpu.async_copy(rows_v[b], quant_hbm.at[pl.ds(base, CH)],
                             sem_s[b]),
        ]
        pltpu.sync_copy(xev[b], wsum_sh.at[ir_v[b]], add=True)
    for c in stores:
        c.wait()

    plsc.subcore_barrier()

    @pl.when(sid == 0)
    def _flush():
        pltpu.sync_copy(wsum_sh, wsum_out.at[cid])


def _finalize_body(wsum_ref, emac_ref, emaw_ref,
                   cnt_out, wgt_out, emb_out, *, batch_size):
    counts = wsum_ref[0, :, D:D + 1] + wsum_ref[1, :, D:D + 1]
    sums = wsum_ref[0, :, :D] + wsum_ref[1, :, :D]
    nc = emac_ref[...] * DECAY + counts * (1.0 - DECAY)
    nc = (nc + EPSILON) / (batch_size + K * EPSILON) * batch_size
    nw = emaw_ref[...] * DECAY + sums * (1.0 - DECAY)
    cnt_out[...] = nc
    wgt_out[...] = nw
    emb_out[...] = nw / nc


def kernel(x, codebook, ema_count, ema_weight):
    batch_size = x.shape[0]
    x_flat = x.reshape(-1, D)
    N = x_flat.shape[0]
    T = 4096
    nblocks = N // T
    per_w = N // NW
    nch = per_w // CH

    tc = functools.partial(_tc_body, T=T)
    disc, idxc, xe = pl.pallas_call(
        tc,
        grid=(nblocks,),
        in_specs=[
            pl.BlockSpec((T, D), lambda i: (i, 0)),
            pl.BlockSpec((D, K), lambda i: (0, 0)),
        ],
        out_specs=[
            pl.BlockSpec((T, K), lambda i: (i, 0)),
            pl.BlockSpec((T, 1), lambda i: (i, 0)),
            pl.BlockSpec((T, W), lambda i: (i, 0)),
        ],
        out_shape=[
            jax.ShapeDtypeStruct((N, K), jnp.float32),
            jax.ShapeDtypeStruct((N, 1), jnp.int32),
            jax.ShapeDtypeStruct((N, W), jnp.float32),
        ],
    )(x_flat, codebook.T)

    idx1 = idxc.reshape(N)
    cb_pad = jnp.concatenate(
        [codebook, jnp.zeros((K, W - D), jnp.float32)], axis=1)
    zw = jnp.zeros((K, W), jnp.float32)

    mesh = plsc.VectorSubcoreMesh(core_axis_name="c", subcore_axis_name="s")
    sc = functools.partial(_sc_body, nch=nch, per_w=per_w)
    sc_kernel = pl.kernel(
        sc,
        out_type=[
            jax.ShapeDtypeStruct((N, W), jnp.float32),
            jax.ShapeDtypeStruct((NC, K, W), jnp.float32),
        ],
        mesh=mesh,
        scratch_types=[
            [pltpu.VMEM((CH,), jnp.int32)] * 2,
            [pltpu.VMEM((CH, W), jnp.float32)] * 2,
            [pltpu.VMEM((CH, W), jnp.float32)] * 2,
            pltpu.VMEM_SHARED((K, W), jnp.float32),
            [pltpu.SemaphoreType.DMA] * 2,
            [pltpu.SemaphoreType.DMA] * 2,
            [pltpu.SemaphoreType.DMA] * 2,
        ],
    )
    quantp, wsum_p = sc_kernel(xe, cb_pad, idx1, zw)

    fin = functools.partial(_finalize_body, batch_size=batch_size)
    new_count, new_weight, new_emb = pl.pallas_call(
        fin,
        out_shape=[
            jax.ShapeDtypeStruct((K, 1), jnp.float32),
            jax.ShapeDtypeStruct((K, D), jnp.float32),
            jax.ShapeDtypeStruct((K, D), jnp.float32),
        ],
    )(wsum_p, ema_count.reshape(K, 1), ema_weight)

    quantized = quantp[:, :D].reshape(x.shape)
    return (disc, quantized, new_count.reshape(K), new_weight, new_emb)


# speedup vs baseline: 1.0070x; 1.0070x over previous
import functools

import jax
import jax.numpy as jnp
from jax import lax
from jax.experimental import pallas as pl
from jax.experimental.pallas import tpu as pltpu
from jax.experimental.pallas import tpu_sc as plsc

K = 1024
D = 64
DECAY = 0.99
EPSILON = 1e-05

NC = 2
NS = 16
NW = NC * NS
CH = 128
W = 128


def _tc_body(x_ref, cbt_ref, disc_ref, idx_ref, xe_ref, *, T):
    xb = x_ref[...]
    cbt = cbt_ref[...]

    x2 = jnp.sum(xb * xb, axis=1, keepdims=True)
    c2 = jnp.sum(cbt * cbt, axis=0, keepdims=True)
    dot = jax.lax.dot_general(xb, cbt, (((1,), (0,)), ((), ())),
                              preferred_element_type=jnp.float32)
    d = (x2 + (-2.0) * dot) + c2

    m = jnp.min(d, axis=1, keepdims=True)
    lane = jax.lax.broadcasted_iota(jnp.int32, (T, K), 1)
    cand = jnp.where(d == m, lane, K)
    idxc = jnp.min(cand, axis=1)[:, None]
    idx_ref[...] = idxc
    disc_ref[...] = (jax.lax.broadcasted_iota(jnp.int32, (T, K), 1)
                     == idxc).astype(jnp.float32)
    ones = jnp.ones((T, 1), jnp.float32)
    zeros = jnp.zeros((T, W - D - 1), jnp.float32)
    xe_ref[...] = jnp.concatenate([xb, ones, zeros], axis=1)


def _sc_body(xe_hbm, cb_hbm, idx_hbm, zw_hbm,
             quant_hbm, wsum_out,
             ir_v, xev, rows_v,
             wsum_sh, sem_l, sem_g, sem_s, sem_a, *, nch, per_w):
    cid = lax.axis_index("c")
    sid = lax.axis_index("s")
    wid = sid * NC + cid

    @pl.when(sid == 0)
    def _init():
        pltpu.sync_copy(zw_hbm, wsum_sh)

    plsc.subcore_barrier()

    def fire_loads(j):
        s = j % 3
        base = wid * per_w + j * CH
        return [
            pltpu.async_copy(idx_hbm.at[pl.ds(base, CH)], ir_v[s], sem_l[s]),
            pltpu.async_copy(xe_hbm.at[pl.ds(base, CH)], xev[s], sem_l[s]),
        ]

    def fire_gather(j):
        return pltpu.async_copy(cb_hbm.at[ir_v[j % 3]], rows_v[j % 2],
                                sem_g[j % 2])

    pending = {0: fire_loads(0)}
    for c in pending.pop(0):
        c.wait()
    gath = {0: fire_gather(0)}
    if nch > 1:
        pending[1] = fire_loads(1)
    stores = {}
    adds = {}
    for j in range(nch):
        s3, s2 = j % 3, j % 2
        base = wid * per_w + j * CH
        gath.pop(j).wait()
        stores[j] = pltpu.async_copy(
            rows_v[s2], quant_hbm.at[pl.ds(base, CH)], sem_s[s2])
        adds[j] = pltpu.async_copy(
            xev[s3], wsum_sh.at[ir_v[s3]], sem_a[s3], add=True)
        if j + 1 < nch:
            for c in pending.pop(j + 1):
                c.wait()
            if (j - 1) in stores:
                stores.pop(j - 1).wait()
            gath[j + 1] = fire_gather(j + 1)
            if j + 2 < nch:
                if (j - 1) in adds:
                    adds.pop(j - 1).wait()
                pending[j + 2] = fire_loads(j + 2)
    for c in stores.values():
        c.wait()
    for c in adds.values():
        c.wait()

    plsc.subcore_barrier()

    @pl.when(sid == 0)
    def _flush():
        pltpu.sync_copy(wsum_sh, wsum_out.at[cid])


def _finalize_body(wsum_ref, emac_ref, emaw_ref,
                   cnt_out, wgt_out, emb_out, *, batch_size):
    counts = wsum_ref[0, :, D:D + 1] + wsum_ref[1, :, D:D + 1]
    sums = wsum_ref[0, :, :D] + wsum_ref[1, :, :D]
    nc = emac_ref[...] * DECAY + counts * (1.0 - DECAY)
    nc = (nc + EPSILON) / (batch_size + K * EPSILON) * batch_size
    nw = emaw_ref[...] * DECAY + sums * (1.0 - DECAY)
    cnt_out[...] = nc
    wgt_out[...] = nw
    emb_out[...] = nw / nc


def kernel(x, codebook, ema_count, ema_weight):
    batch_size = x.shape[0]
    x_flat = x.reshape(-1, D)
    N = x_flat.shape[0]
    T = 2048
    nblocks = N // T
    per_w = N // NW
    nch = per_w // CH

    tc = functools.partial(_tc_body, T=T)
    disc, idxc, xe = pl.pallas_call(
        tc,
        grid=(nblocks,),
        in_specs=[
            pl.BlockSpec((T, D), lambda i: (i, 0)),
            pl.BlockSpec((D, K), lambda i: (0, 0)),
        ],
        out_specs=[
            pl.BlockSpec((T, K), lambda i: (i, 0)),
            pl.BlockSpec((T, 1), lambda i: (i, 0)),
            pl.BlockSpec((T, W), lambda i: (i, 0)),
        ],
        out_shape=[
            jax.ShapeDtypeStruct((N, K), jnp.float32),
            jax.ShapeDtypeStruct((N, 1), jnp.int32),
            jax.ShapeDtypeStruct((N, W), jnp.float32),
        ],
    )(x_flat, codebook.T)

    idx1 = idxc.reshape(N)
    cb_pad = jnp.concatenate(
        [codebook, jnp.zeros((K, W - D), jnp.float32)], axis=1)
    zw = jnp.zeros((K, W), jnp.float32)

    mesh = plsc.VectorSubcoreMesh(core_axis_name="c", subcore_axis_name="s")
    sc = functools.partial(_sc_body, nch=nch, per_w=per_w)
    sc_kernel = pl.kernel(
        sc,
        out_type=[
            jax.ShapeDtypeStruct((N, W), jnp.float32),
            jax.ShapeDtypeStruct((NC, K, W), jnp.float32),
        ],
        mesh=mesh,
        scratch_types=[
            [pltpu.VMEM((CH,), jnp.int32)] * 3,
            [pltpu.VMEM((CH, W), jnp.float32)] * 3,
            [pltpu.VMEM((CH, W), jnp.float32)] * 2,
            pltpu.VMEM_SHARED((K, W), jnp.float32),
            [pltpu.SemaphoreType.DMA] * 3,
            [pltpu.SemaphoreType.DMA] * 2,
            [pltpu.SemaphoreType.DMA] * 2,
            [pltpu.SemaphoreType.DMA] * 3,
        ],
    )
    quantp, wsum_p = sc_kernel(xe, cb_pad, idx1, zw)

    fin = functools.partial(_finalize_body, batch_size=batch_size)
    new_count, new_weight, new_emb = pl.pallas_call(
        fin,
        out_shape=[
            jax.ShapeDtypeStruct((K, 1), jnp.float32),
            jax.ShapeDtypeStruct((K, D), jnp.float32),
            jax.ShapeDtypeStruct((K, D), jnp.float32),
        ],
    )(wsum_p, ema_count.reshape(K, 1), ema_weight)

    quantized = quantp[:, :D].reshape(x.shape)
    return (disc, quantized, new_count.reshape(K), new_weight, new_emb)
